# baseline (speedup 1.0000x reference)
import jax
import jax.numpy as jnp
from jax.experimental import pallas as pl
from jax.experimental.pallas import tpu as pltpu

_NCHUNK = 8


def _dma_copy_body(x_hbm, o_hbm, sems):
    b = x_hbm.shape[0]
    chunk = b // _NCHUNK
    copies = [
        pltpu.make_async_copy(
            x_hbm.at[pl.ds(i * chunk, chunk)],
            o_hbm.at[pl.ds(i * chunk, chunk)],
            sems.at[i],
        )
        for i in range(_NCHUNK)
    ]
    for c in copies:
        c.start()
    for c in copies:
        c.wait()


def kernel(X):
    B, S, F = X.shape
    return pl.pallas_call(
        _dma_copy_body,
        in_specs=[pl.BlockSpec(memory_space=pl.ANY)],
        out_specs=pl.BlockSpec(memory_space=pl.ANY),
        scratch_shapes=[pltpu.SemaphoreType.DMA((_NCHUNK,))],
        out_shape=jax.ShapeDtypeStruct((B, S, F), jnp.float32),
    )(X)


# speedup vs baseline: 13.2903x; 13.2903x over previous
import jax
import jax.numpy as jnp
from jax.experimental import pallas as pl
from jax.experimental.pallas import tpu as pltpu

_BLK = 16
_K = 8
_L = 3


def _make_body(B, S, F):
    nblk = B // _BLK

    def body(x_hbm, o_hbm, *scratch):
        bufs = scratch[:_K]
        in_sems, out_sems = scratch[_K], scratch[_K + 1]

        def in_copy(i):
            s = i % _K
            return pltpu.make_async_copy(
                x_hbm.at[pl.ds(i * _BLK, _BLK)], bufs[s], in_sems.at[s])

        def out_copy(i):
            s = i % _K
            return pltpu.make_async_copy(
                bufs[s], o_hbm.at[pl.ds(i * _BLK, _BLK)], out_sems.at[s])

        for k in range(min(_L, nblk)):
            in_copy(k).start()
        for i in range(nblk):
            j = i + _L
            if j < nblk:
                if j - _K >= 0:
                    out_copy(j - _K).wait()
                in_copy(j).start()
            in_copy(i).wait()
            out_copy(i).start()
        for i in range(max(0, nblk - _K), nblk):
            out_copy(i).wait()

    return body, nblk


def kernel(X):
    B, S, F = X.shape
    body, _ = _make_body(B, S, F)
    scratch = [pltpu.VMEM((_BLK, S, F), jnp.float32) for _ in range(_K)]
    scratch += [pltpu.SemaphoreType.DMA((_K,)), pltpu.SemaphoreType.DMA((_K,))]
    return pl.pallas_call(
        body,
        in_specs=[pl.BlockSpec(memory_space=pl.ANY)],
        out_specs=pl.BlockSpec(memory_space=pl.ANY),
        scratch_shapes=scratch,
        out_shape=jax.ShapeDtypeStruct((B, S, F), jnp.float32),
    )(X)
